# baseline (speedup 1.0000x reference)
import functools

import jax
import jax.numpy as jnp
from jax import lax
from jax.experimental import pallas as pl
from jax.experimental.pallas import tpu as pltpu
from jax.experimental.pallas import tpu_sc as plsc

NV = 10000
EV = 320000
NE = 320000
ET = 640000

NC = 2
NS = 16
LANES = 16
NW = NC * NS
CH = 128
JUNK = 16

_i32 = jnp.int32
_f32 = jnp.float32

_MESH = plsc.VectorSubcoreMesh(core_axis_name="c", subcore_axis_name="s")


def _lane_iota():
  return lax.iota(_i32, LANES)


STEP = 624
SZ = 640
STEPB = 312
SZB = 320
JUNKB = 80


def _slab(t):
  return pl.ds(t * STEP, SZ)


DG = 4
SG = 8


def _copy_idx(dst_ded, src_big, pos):
  for jj in range(CH // LANES):
    sl = pl.ds(jj * LANES, LANES)
    dst_ded[sl] = src_big[pl.ds(pos + jj * LANES, LANES)]


def _sc_agg_binned(n, d, r, p, with_gather):
  assert n == r * p and p % NC == 0 and STEPB * (NS - 1) + SZB == r
  ppc = p // NC
  width = d if with_gather else 128
  n_init = -(-SZB // CH)

  scratch = [pltpu.VMEM((SG * CH,), _i32), pltpu.VMEM((SG * CH,), _i32)]
  scratch += [pltpu.VMEM((CH,), _i32) for _ in range(2 * DG)]
  scratch += [
      pltpu.VMEM((DG, CH, width) if with_gather else (CH, width), _f32),
      pltpu.VMEM((96,), _i32),
      pltpu.VMEM_SHARED((r + JUNKB, width), _f32),
      pltpu.SemaphoreType.DMA,
      pltpu.SemaphoreType.DMA,
  ]

  def body(*refs):
    if with_gather:
      g_hbm, se_hbm, de_hbm, bnd_hbm, out_hbm = refs[:5]
    else:
      se_hbm, de_hbm, bnd_hbm, ones_hbm, out_hbm = refs[:5]
    scr = refs[5:]
    sbig, dbig = scr[0], scr[1]
    sded = scr[2:2 + DG]
    dded = scr[2 + DG:2 + 2 * DG]
    rows_v, bnd_v, acc, sem_g, sem_s = scr[2 + 2 * DG:]
    c = lax.axis_index("c")
    t = lax.axis_index("s")
    w = c * NS + t
    iot = _lane_iota()
    pltpu.sync_copy(bnd_hbm, bnd_v)
    if not with_gather:
      pltpu.sync_copy(ones_hbm, rows_v)

    def one_pass(j, carry):
      pw = c * ppc + j
      base = pw * r
      bv = bnd_v[pl.ds(pw, LANES)]
      start = bv[0]
      end = bv[1]
      if with_gather:
        pltpu.sync_copy(g_hbm.at[pl.ds(base + t * STEPB, SZB)],
                        acc.at[pl.ds(t * STEPB, SZB)])
      else:
        for rr in range(n_init):
          pltpu.sync_copy(rows_v, acc.at[pl.ds(t * STEPB + rr * CH, CH)])
      plsc.subcore_barrier()

      sa_row = start // CH
      trip = (end - sa_row * CH + CH - 1) // CH
      q = (trip + NS - 1) // NS
      cc_beg = t * q
      cc_end = jnp.minimum((t + 1) * q, trip)
      n_sg = (q + SG - 1) // SG

      def sgroup(gg, carry2):
        sgrow = cc_beg + gg * SG
        off = (sa_row + sgrow) * CH
        pltpu.sync_copy(de_hbm.at[pl.ds(off, SG * CH)], dbig)
        if with_gather:
          pltpu.sync_copy(se_hbm.at[pl.ds(off, SG * CH)], sbig)
        for sub in range(SG // DG):
          for b in range(DG):
            cc = sgrow + sub * DG + b
            end_c = jnp.where(cc < cc_end, end, -1)
            pos = (sub * DG + b) * CH
            for jj in range(CH // LANES):
              ge = (sa_row + cc) * CH + jj * LANES + iot
              valid = (ge >= start) & (ge < end_c)
              sl = pl.ds(jj * LANES, LANES)
              psl = pl.ds(pos + jj * LANES, LANES)
              dded[b][sl] = jnp.where(valid, dbig[psl], r + iot)
              if with_gather:
                sded[b][sl] = jnp.where(valid, sbig[psl], w * LANES + iot)
          if with_gather:
            gathers = [pltpu.async_copy(g_hbm.at[sded[b]], rows_v.at[b],
                                        sem_g) for b in range(DG)]
            for gth in gathers:
              gth.wait()
            scatters = [pltpu.async_copy(rows_v.at[b], acc.at[dded[b]],
                                         sem_s, add=True)
                        for b in range(DG)]
          else:
            scatters = [pltpu.async_copy(rows_v, acc.at[dded[b]], sem_s,
                                         add=True) for b in range(DG)]
          for sct in scatters:
            sct.wait()
        return carry2

      lax.fori_loop(0, n_sg, sgroup, 0)
      plsc.subcore_barrier()
      pltpu.sync_copy(acc.at[pl.ds(t * STEPB, SZB)],
                      out_hbm.at[pl.ds(base + t * STEPB, SZB)])
      plsc.subcore_barrier()
      return carry

    lax.fori_loop(0, ppc, one_pass, 0)

  return functools.partial(
      pl.kernel,
      out_type=jax.ShapeDtypeStruct((n, width), _f32),
      mesh=_MESH,
      scratch_types=scratch,
  )(body)


_BR = 2000


def _tc(body, n, dout, in_arrays, in_shapes):
  specs = []
  for s in in_shapes:
    if s[0] is None:
      specs.append(pl.BlockSpec(s[1], lambda i: (0, 0)))
    else:
      specs.append(pl.BlockSpec((_BR, s[1]), lambda i: (i, 0)))
  return pl.pallas_call(
      body,
      grid=(n // _BR,),
      in_specs=specs,
      out_specs=pl.BlockSpec((_BR, dout), lambda i: (i, 0)),
      out_shape=jax.ShapeDtypeStruct((n, dout), _f32),
  )(*in_arrays)


def _mm(x, w, dinv, pad_to=None):
  n, kdim = x.shape
  dout = w.shape[1]
  width = pad_to or dout

  def body(x_ref, w_ref, d_ref, o_ref):
    y = jnp.dot(x_ref[...], w_ref[...],
                preferred_element_type=_f32) * d_ref[...]
    if width > dout:
      y = jnp.concatenate([y, jnp.zeros((y.shape[0], width - dout), _f32)],
                          axis=1)
    o_ref[...] = y

  return _tc(body, n, width, (x, w, dinv),
             ((0, kdim), (None, (kdim, dout)), (0, 1)))


def _post(a, dinv, b):
  n, d = a.shape

  def body(a_ref, d_ref, b_ref, o_ref):
    o_ref[...] = jnp.maximum(a_ref[...] * d_ref[...] + b_ref[...], 0.0)

  return _tc(body, n, d, (a, dinv, b), ((0, d), (0, 1), (None, (1, d))))


def _post_mm(a, dinv, b, w2, real_d=None, pad_to=None):
  n, d = a.shape
  rd = real_d or d
  kd = w2.shape[0]
  dout = w2.shape[1]
  width = pad_to or dout

  def body(a_ref, d_ref, b_ref, w_ref, o_ref):
    h = jnp.maximum(a_ref[...][:, :rd] * d_ref[...] + b_ref[...], 0.0)
    y = jnp.dot(h, w_ref[...], preferred_element_type=_f32) * d_ref[...]
    if width > dout:
      y = jnp.concatenate([y, jnp.zeros((y.shape[0], width - dout), _f32)],
                          axis=1)
    o_ref[...] = y

  return _tc(body, n, width, (a, dinv, b, w2),
             ((0, d), (0, 1), (None, (1, rd)), (None, (kd, dout))))


def _out2(a_first, b_first, a_second, b_second, dinv, real_d1=None):
  n, d1 = a_first.shape
  rd1 = real_d1 or d1
  d2 = a_second.shape[1]

  def body(a1_ref, b1_ref, a2_ref, b2_ref, d_ref, o_ref):
    y1 = jnp.maximum(a1_ref[...][:, :rd1] * d_ref[...] + b1_ref[...], 0.0)
    y2 = jnp.maximum(a2_ref[...] * d_ref[...] + b2_ref[...], 0.0)
    o_ref[...] = jnp.concatenate([y1, y2], axis=1)

  return _tc(body, n, rd1 + d2, (a_first, b_first, a_second, b_second, dinv),
             ((0, d1), (None, (1, rd1)), (0, d2), (None, (1, d2)), (0, 1)))


def _dinv1(deg):
  n = deg.shape[0]

  def body(deg_ref, o_ref):
    o_ref[...] = lax.rsqrt(jnp.maximum(deg_ref[...][:, :1], 1e-12))

  return _tc(body, n, 1, (deg,), ((0, 128),))


def kernel(feature_v, edge_index, feature_e, trans_edge_index,
           W1v, b1v, W1e, b1e, Ws1, bs1, Ws2, bs2,
           W2v, b2v, W3v, b3v, W2e, b2e, W3e, b3e):
  ei = edge_index.astype(_i32)
  te = trans_edge_index.astype(_i32)
  src_v, dst_v = ei[0], ei[1]
  src_e, dst_e = te[0], te[1]

  rv = 5000
  pv = NV // rv
  bucket_v = dst_v // rv
  order_v = jnp.argsort(bucket_v)
  sv = src_v[order_v]
  dstv_s = dst_v[order_v]
  dvl = dstv_s - (dstv_s // rv) * rv
  bnd_v = jnp.searchsorted(dstv_s // rv,
                           jnp.arange(pv + 1, dtype=_i32)).astype(_i32)
  bnd_v96 = jnp.concatenate([bnd_v, jnp.full((96 - (pv + 1),), EV, _i32)])
  erv = EV // CH + 64
  padv = erv * CH - EV
  sv_p = jnp.concatenate([sv, jnp.zeros((padv,), _i32)])
  dvl_p = jnp.concatenate([dvl, jnp.full((padv,), rv, _i32)])

  r = 5000
  p = NE // r
  bucket = dst_e // r
  order = jnp.argsort(bucket)
  se = src_e[order]
  dst_s = dst_e[order]
  de = dst_s - (dst_s // r) * r
  bnd = jnp.searchsorted(dst_s // r,
                         jnp.arange(p + 1, dtype=_i32)).astype(_i32)
  bnd64 = jnp.concatenate([bnd, jnp.full((96 - (p + 1),), ET, _i32)])
  er = ET // CH + 64
  padl = er * CH - ET
  se_p = jnp.concatenate([se, jnp.zeros((padl,), _i32)])
  de_p = jnp.concatenate([de, jnp.full((padl,), r, _i32)])
  ones_sc = jnp.ones((CH, 128), _f32)

  b1v_ = b1v.reshape(1, -1)
  b1e_ = b1e.reshape(1, -1)
  bs1_ = bs1.reshape(1, -1)
  bs2_ = bs2.reshape(1, -1)
  b2v_ = b2v.reshape(1, -1)
  b3v_ = b3v.reshape(1, -1)
  b2e_ = b2e.reshape(1, -1)
  b3e_ = b3e.reshape(1, -1)

  agg_v = _sc_agg_binned(NV, 128, rv, pv, True)
  deg_v_k = _sc_agg_binned(NV, 0, rv, pv, False)
  agg_e128 = _sc_agg_binned(NE, 128, r, p, True)
  deg_e_k = _sc_agg_binned(NE, 0, r, p, False)

  degv = deg_v_k(sv_p, dvl_p, bnd_v96, ones_sc)
  dege = deg_e_k(se_p, de_p, bnd64, ones_sc)
  dinv_v = _dinv1(degv)
  dinv_e = _dinv1(dege)

  g1 = _mm(feature_v, W1v, dinv_v)
  A1 = agg_v(g1, sv_p, dvl_p, bnd_v96)
  fv = _post(A1, dinv_v, b1v_)

  gs1 = _mm(fv, Ws1, dinv_v)
  As1 = agg_v(gs1, sv_p, dvl_p, bnd_v96)
  gs2 = _post_mm(As1, dinv_v, bs1_, Ws2)
  As2 = agg_v(gs2, sv_p, dvl_p, bnd_v96)

  g2 = _mm(fv, W2v, dinv_v)
  A2 = agg_v(g2, sv_p, dvl_p, bnd_v96)
  g3 = _post_mm(A2, dinv_v, b2v_, W3v)
  A3 = agg_v(g3, sv_p, dvl_p, bnd_v96)

  fv_out = _out2(A3, b3v_, As2, bs2_, dinv_v)

  ge1 = _mm(feature_e, W1e, dinv_e)
  Ae1 = agg_e128(ge1, se_p, de_p, bnd64)
  fe = _post(Ae1, dinv_e, b1e_)

  ges1 = _mm(fe, Ws1, dinv_e)
  Aes1 = agg_e128(ges1, se_p, de_p, bnd64)
  ges2 = _post_mm(Aes1, dinv_e, bs1_, Ws2)
  Aes2 = agg_e128(ges2, se_p, de_p, bnd64)

  ge2 = _mm(fe, W2e, dinv_e, pad_to=128)
  Ae2 = agg_e128(ge2, se_p, de_p, bnd64)
  ge3 = _post_mm(Ae2, dinv_e, b2e_, W3e, real_d=64, pad_to=128)
  Ae3 = agg_e128(ge3, se_p, de_p, bnd64)

  fe_out = _out2(Ae3, b3e_, Aes2, bs2_, dinv_e, real_d1=64)

  return fv_out, fe_out


# speedup vs baseline: 1.2091x; 1.2091x over previous
import functools

import jax
import jax.numpy as jnp
from jax import lax
from jax.experimental import pallas as pl
from jax.experimental.pallas import tpu as pltpu
from jax.experimental.pallas import tpu_sc as plsc

NV = 10000
EV = 320000
NE = 320000
ET = 640000

NC = 2
NS = 16
LANES = 16
NW = NC * NS
CH = 128
JUNK = 16

_i32 = jnp.int32
_f32 = jnp.float32

_MESH = plsc.VectorSubcoreMesh(core_axis_name="c", subcore_axis_name="s")


def _lane_iota():
  return lax.iota(_i32, LANES)


STEP = 624
SZ = 640
STEPB = 312
SZB = 320
JUNKB = 80


def _slab(t):
  return pl.ds(t * STEP, SZ)


DG = 4
SG = 8


def _copy_idx(dst_ded, src_big, pos):
  for jj in range(CH // LANES):
    sl = pl.ds(jj * LANES, LANES)
    dst_ded[sl] = src_big[pl.ds(pos + jj * LANES, LANES)]


def _sc_agg_binned(n, d, r, p, with_gather):
  assert n == r * p and p % NC == 0 and STEPB * (NS - 1) + SZB == r
  ppc = p // NC
  width = d if with_gather else 128
  n_init = -(-SZB // CH)

  scratch = [pltpu.VMEM((SG * CH,), _i32), pltpu.VMEM((SG * CH,), _i32)]
  scratch += [pltpu.VMEM((CH,), _i32) for _ in range(2 * DG)]
  scratch += [
      pltpu.VMEM((DG, CH, width) if with_gather else (CH, width), _f32),
      pltpu.VMEM((96,), _i32),
      pltpu.VMEM_SHARED((r + JUNKB, width), _f32),
      pltpu.SemaphoreType.DMA,
      pltpu.SemaphoreType.DMA,
  ]

  def body(*refs):
    if with_gather:
      g_hbm, se_hbm, de_hbm, bnd_hbm, out_hbm = refs[:5]
    else:
      se_hbm, de_hbm, bnd_hbm, ones_hbm, out_hbm = refs[:5]
    scr = refs[5:]
    sbig, dbig = scr[0], scr[1]
    sded = scr[2:2 + DG]
    dded = scr[2 + DG:2 + 2 * DG]
    rows_v, bnd_v, acc, sem_g, sem_s = scr[2 + 2 * DG:]
    c = lax.axis_index("c")
    t = lax.axis_index("s")
    w = c * NS + t
    iot = _lane_iota()
    pltpu.sync_copy(bnd_hbm, bnd_v)
    if not with_gather:
      pltpu.sync_copy(ones_hbm, rows_v)

    def one_pass(j, carry):
      pw = c * ppc + j
      base = pw * r
      bv = bnd_v[pl.ds(pw, LANES)]
      start = bv[0]
      end = bv[1]
      if with_gather:
        pltpu.sync_copy(g_hbm.at[pl.ds(base + t * STEPB, SZB)],
                        acc.at[pl.ds(t * STEPB, SZB)])
      else:
        for rr in range(n_init):
          pltpu.sync_copy(rows_v, acc.at[pl.ds(t * STEPB + rr * CH, CH)])
      plsc.subcore_barrier()

      sa_row = start // CH
      trip = (end - sa_row * CH + CH - 1) // CH
      q = (trip + NS - 1) // NS
      cc_beg = t * q
      cc_end = jnp.minimum((t + 1) * q, trip)
      n_sg = (q + SG - 1) // SG

      def sgroup(gg, carry2):
        sgrow = cc_beg + gg * SG
        off = (sa_row + sgrow) * CH
        pltpu.sync_copy(de_hbm.at[pl.ds(off, SG * CH)], dbig)
        if with_gather:
          pltpu.sync_copy(se_hbm.at[pl.ds(off, SG * CH)], sbig)
        for sub in range(SG // DG):
          conds = []
          for b in range(DG):
            cc = sgrow + sub * DG + b
            conds.append(cc < cc_end)
            pos = (sub * DG + b) * CH
            for jj in range(CH // LANES):
              ge = (sa_row + cc) * CH + jj * LANES + iot
              valid = (ge >= start) & (ge < end)
              sl = pl.ds(jj * LANES, LANES)
              psl = pl.ds(pos + jj * LANES, LANES)
              dded[b][sl] = jnp.where(valid, dbig[psl], r + iot)
              if with_gather:
                sded[b][sl] = jnp.where(valid, sbig[psl], w * LANES + iot)
          if with_gather:
            gds = [pltpu.make_async_copy(g_hbm.at[sded[b]], rows_v.at[b],
                                         sem_g) for b in range(DG)]
            for b in range(DG):
              pl.when(conds[b])(lambda b=b: gds[b].start())
            for b in range(DG):
              pl.when(conds[b])(lambda b=b: gds[b].wait())
            sds = [pltpu.make_async_copy(rows_v.at[b], acc.at[dded[b]],
                                         sem_s) for b in range(DG)]
          else:
            sds = [pltpu.make_async_copy(rows_v, acc.at[dded[b]], sem_s)
                   for b in range(DG)]
          for b in range(DG):
            pl.when(conds[b])(lambda b=b: sds[b].start(add=True))
          for b in range(DG):
            pl.when(conds[b])(lambda b=b: sds[b].wait())
        return carry2

      lax.fori_loop(0, n_sg, sgroup, 0)
      plsc.subcore_barrier()
      pltpu.sync_copy(acc.at[pl.ds(t * STEPB, SZB)],
                      out_hbm.at[pl.ds(base + t * STEPB, SZB)])
      plsc.subcore_barrier()
      return carry

    lax.fori_loop(0, ppc, one_pass, 0)

  return functools.partial(
      pl.kernel,
      out_type=jax.ShapeDtypeStruct((n, width), _f32),
      mesh=_MESH,
      scratch_types=scratch,
  )(body)


_BR = 2000


def _tc(body, n, dout, in_arrays, in_shapes):
  specs = []
  for s in in_shapes:
    if s[0] is None:
      specs.append(pl.BlockSpec(s[1], lambda i: (0, 0)))
    else:
      specs.append(pl.BlockSpec((_BR, s[1]), lambda i: (i, 0)))
  return pl.pallas_call(
      body,
      grid=(n // _BR,),
      in_specs=specs,
      out_specs=pl.BlockSpec((_BR, dout), lambda i: (i, 0)),
      out_shape=jax.ShapeDtypeStruct((n, dout), _f32),
  )(*in_arrays)


def _mm(x, w, dinv, pad_to=None):
  n, kdim = x.shape
  dout = w.shape[1]
  width = pad_to or dout

  def body(x_ref, w_ref, d_ref, o_ref):
    y = jnp.dot(x_ref[...], w_ref[...],
                preferred_element_type=_f32) * d_ref[...]
    if width > dout:
      y = jnp.concatenate([y, jnp.zeros((y.shape[0], width - dout), _f32)],
                          axis=1)
    o_ref[...] = y

  return _tc(body, n, width, (x, w, dinv),
             ((0, kdim), (None, (kdim, dout)), (0, 1)))


def _post(a, dinv, b):
  n, d = a.shape

  def body(a_ref, d_ref, b_ref, o_ref):
    o_ref[...] = jnp.maximum(a_ref[...] * d_ref[...] + b_ref[...], 0.0)

  return _tc(body, n, d, (a, dinv, b), ((0, d), (0, 1), (None, (1, d))))


def _post_mm(a, dinv, b, w2, real_d=None, pad_to=None):
  n, d = a.shape
  rd = real_d or d
  kd = w2.shape[0]
  dout = w2.shape[1]
  width = pad_to or dout

  def body(a_ref, d_ref, b_ref, w_ref, o_ref):
    h = jnp.maximum(a_ref[...][:, :rd] * d_ref[...] + b_ref[...], 0.0)
    y = jnp.dot(h, w_ref[...], preferred_element_type=_f32) * d_ref[...]
    if width > dout:
      y = jnp.concatenate([y, jnp.zeros((y.shape[0], width - dout), _f32)],
                          axis=1)
    o_ref[...] = y

  return _tc(body, n, width, (a, dinv, b, w2),
             ((0, d), (0, 1), (None, (1, rd)), (None, (kd, dout))))


def _out2(a_first, b_first, a_second, b_second, dinv, real_d1=None):
  n, d1 = a_first.shape
  rd1 = real_d1 or d1
  d2 = a_second.shape[1]

  def body(a1_ref, b1_ref, a2_ref, b2_ref, d_ref, o_ref):
    y1 = jnp.maximum(a1_ref[...][:, :rd1] * d_ref[...] + b1_ref[...], 0.0)
    y2 = jnp.maximum(a2_ref[...] * d_ref[...] + b2_ref[...], 0.0)
    o_ref[...] = jnp.concatenate([y1, y2], axis=1)

  return _tc(body, n, rd1 + d2, (a_first, b_first, a_second, b_second, dinv),
             ((0, d1), (None, (1, rd1)), (0, d2), (None, (1, d2)), (0, 1)))


def _dinv1(deg):
  n = deg.shape[0]

  def body(deg_ref, o_ref):
    o_ref[...] = lax.rsqrt(jnp.maximum(deg_ref[...][:, :1], 1e-12))

  return _tc(body, n, 1, (deg,), ((0, 128),))


def kernel(feature_v, edge_index, feature_e, trans_edge_index,
           W1v, b1v, W1e, b1e, Ws1, bs1, Ws2, bs2,
           W2v, b2v, W3v, b3v, W2e, b2e, W3e, b3e):
  ei = edge_index.astype(_i32)
  te = trans_edge_index.astype(_i32)
  src_v, dst_v = ei[0], ei[1]
  src_e, dst_e = te[0], te[1]

  rv = 5000
  pv = NV // rv
  bucket_v = dst_v // rv
  order_v = jnp.argsort(bucket_v)
  sv = src_v[order_v]
  dstv_s = dst_v[order_v]
  dvl = dstv_s - (dstv_s // rv) * rv
  bnd_v = jnp.searchsorted(dstv_s // rv,
                           jnp.arange(pv + 1, dtype=_i32)).astype(_i32)
  bnd_v96 = jnp.concatenate([bnd_v, jnp.full((96 - (pv + 1),), EV, _i32)])
  erv = EV // CH + 64
  padv = erv * CH - EV
  sv_p = jnp.concatenate([sv, jnp.zeros((padv,), _i32)])
  dvl_p = jnp.concatenate([dvl, jnp.full((padv,), rv, _i32)])

  r = 5000
  p = NE // r
  bucket = dst_e // r
  order = jnp.argsort(bucket)
  se = src_e[order]
  dst_s = dst_e[order]
  de = dst_s - (dst_s // r) * r
  bnd = jnp.searchsorted(dst_s // r,
                         jnp.arange(p + 1, dtype=_i32)).astype(_i32)
  bnd64 = jnp.concatenate([bnd, jnp.full((96 - (p + 1),), ET, _i32)])
  er = ET // CH + 64
  padl = er * CH - ET
  se_p = jnp.concatenate([se, jnp.zeros((padl,), _i32)])
  de_p = jnp.concatenate([de, jnp.full((padl,), r, _i32)])
  ones_sc = jnp.ones((CH, 128), _f32)

  b1v_ = b1v.reshape(1, -1)
  b1e_ = b1e.reshape(1, -1)
  bs1_ = bs1.reshape(1, -1)
  bs2_ = bs2.reshape(1, -1)
  b2v_ = b2v.reshape(1, -1)
  b3v_ = b3v.reshape(1, -1)
  b2e_ = b2e.reshape(1, -1)
  b3e_ = b3e.reshape(1, -1)

  agg_v = _sc_agg_binned(NV, 128, rv, pv, True)
  deg_v_k = _sc_agg_binned(NV, 0, rv, pv, False)
  agg_e128 = _sc_agg_binned(NE, 128, r, p, True)
  deg_e_k = _sc_agg_binned(NE, 0, r, p, False)

  degv = deg_v_k(sv_p, dvl_p, bnd_v96, ones_sc)
  dege = deg_e_k(se_p, de_p, bnd64, ones_sc)
  dinv_v = _dinv1(degv)
  dinv_e = _dinv1(dege)

  g1 = _mm(feature_v, W1v, dinv_v)
  A1 = agg_v(g1, sv_p, dvl_p, bnd_v96)
  fv = _post(A1, dinv_v, b1v_)

  gs1 = _mm(fv, Ws1, dinv_v)
  As1 = agg_v(gs1, sv_p, dvl_p, bnd_v96)
  gs2 = _post_mm(As1, dinv_v, bs1_, Ws2)
  As2 = agg_v(gs2, sv_p, dvl_p, bnd_v96)

  g2 = _mm(fv, W2v, dinv_v)
  A2 = agg_v(g2, sv_p, dvl_p, bnd_v96)
  g3 = _post_mm(A2, dinv_v, b2v_, W3v)
  A3 = agg_v(g3, sv_p, dvl_p, bnd_v96)

  fv_out = _out2(A3, b3v_, As2, bs2_, dinv_v)

  ge1 = _mm(feature_e, W1e, dinv_e)
  Ae1 = agg_e128(ge1, se_p, de_p, bnd64)
  fe = _post(Ae1, dinv_e, b1e_)

  ges1 = _mm(fe, Ws1, dinv_e)
  Aes1 = agg_e128(ges1, se_p, de_p, bnd64)
  ges2 = _post_mm(Aes1, dinv_e, bs1_, Ws2)
  Aes2 = agg_e128(ges2, se_p, de_p, bnd64)

  ge2 = _mm(fe, W2e, dinv_e, pad_to=128)
  Ae2 = agg_e128(ge2, se_p, de_p, bnd64)
  ge3 = _post_mm(Ae2, dinv_e, b2e_, W3e, real_d=64, pad_to=128)
  Ae3 = agg_e128(ge3, se_p, de_p, bnd64)

  fe_out = _out2(Ae3, b3e_, Aes2, bs2_, dinv_e, real_d1=64)

  return fv_out, fe_out


# speedup vs baseline: 1.2098x; 1.0006x over previous
import functools

import jax
import jax.numpy as jnp
from jax import lax
from jax.experimental import pallas as pl
from jax.experimental.pallas import tpu as pltpu
from jax.experimental.pallas import tpu_sc as plsc

NV = 10000
EV = 320000
NE = 320000
ET = 640000

NC = 2
NS = 16
LANES = 16
NW = NC * NS
CH = 128

_i32 = jnp.int32
_f32 = jnp.float32

_MESH = plsc.VectorSubcoreMesh(core_axis_name="c", subcore_axis_name="s")


def _lane_iota():
  return lax.iota(_i32, LANES)


STEPB = 312
SZB = 320
JUNKB = 80

DG = 4
SG = 8


def _sc_agg_binned(n, d, r, p, with_gather):
  assert n == r * p and p % NC == 0 and STEPB * (NS - 1) + SZB == r
  ppc = p // NC
  width = d if with_gather else 128
  n_init = -(-SZB // CH)

  scratch = [pltpu.VMEM((SG * CH,), _i32), pltpu.VMEM((SG * CH,), _i32)]
  scratch += [pltpu.VMEM((CH,), _i32) for _ in range(2 * DG)]
  scratch += [
      pltpu.VMEM((DG, CH, width) if with_gather else (CH, width), _f32),
      pltpu.VMEM((96,), _i32),
      pltpu.VMEM_SHARED((r + JUNKB, width), _f32),
      pltpu.SemaphoreType.DMA,
      pltpu.SemaphoreType.DMA,
  ]

  def body(*refs):
    if with_gather:
      g_hbm, se_hbm, de_hbm, bnd_hbm, out_hbm = refs[:5]
    else:
      se_hbm, de_hbm, bnd_hbm, ones_hbm, out_hbm = refs[:5]
    scr = refs[5:]
    sbig, dbig = scr[0], scr[1]
    sded = scr[2:2 + DG]
    dded = scr[2 + DG:2 + 2 * DG]
    rows_v, bnd_v, acc, sem_g, sem_s = scr[2 + 2 * DG:]
    c = lax.axis_index("c")
    t = lax.axis_index("s")
    w = c * NS + t
    iot = _lane_iota()
    pltpu.sync_copy(bnd_hbm, bnd_v)
    if not with_gather:
      pltpu.sync_copy(ones_hbm, rows_v)

    def one_pass(j, carry):
      pw = c * ppc + j
      base = pw * r
      bv = bnd_v[pl.ds(pw, LANES)]
      start = bv[0]
      end = bv[1]
      if with_gather:
        pltpu.sync_copy(g_hbm.at[pl.ds(base + t * STEPB, SZB)],
                        acc.at[pl.ds(t * STEPB, SZB)])
      else:
        for rr in range(n_init):
          pltpu.sync_copy(rows_v, acc.at[pl.ds(t * STEPB + rr * CH, CH)])
      plsc.subcore_barrier()

      sa_row = start // CH
      trip = (end - sa_row * CH + CH - 1) // CH
      q = (trip + NS - 1) // NS
      cc_beg = t * q
      cc_end = jnp.minimum((t + 1) * q, trip)
      n_sg = (q + SG - 1) // SG

      def sgroup(gg, carry2):
        sgrow = cc_beg + gg * SG
        off = (sa_row + sgrow) * CH
        pltpu.sync_copy(de_hbm.at[pl.ds(off, SG * CH)], dbig)
        if with_gather:
          pltpu.sync_copy(se_hbm.at[pl.ds(off, SG * CH)], sbig)
        for sub in range(SG // DG):
          conds = []
          for b in range(DG):
            cc = sgrow + sub * DG + b
            conds.append(cc < cc_end)
            pos = (sub * DG + b) * CH
            for jj in range(CH // LANES):
              ge = (sa_row + cc) * CH + jj * LANES + iot
              valid = (ge >= start) & (ge < end)
              sl = pl.ds(jj * LANES, LANES)
              psl = pl.ds(pos + jj * LANES, LANES)
              dded[b][sl] = jnp.where(valid, dbig[psl], r + iot)
              if with_gather:
                sded[b][sl] = jnp.where(valid, sbig[psl], w * LANES + iot)
          if with_gather:
            gds = [pltpu.make_async_copy(g_hbm.at[sded[b]], rows_v.at[b],
                                         sem_g) for b in range(DG)]
            for b in range(DG):
              pl.when(conds[b])(lambda b=b: gds[b].start())
            for b in range(DG):
              pl.when(conds[b])(lambda b=b: gds[b].wait())
            sds = [pltpu.make_async_copy(rows_v.at[b], acc.at[dded[b]],
                                         sem_s) for b in range(DG)]
          else:
            sds = [pltpu.make_async_copy(rows_v, acc.at[dded[b]], sem_s)
                   for b in range(DG)]
          for b in range(DG):
            pl.when(conds[b])(lambda b=b: sds[b].start(add=True))
          for b in range(DG):
            pl.when(conds[b])(lambda b=b: sds[b].wait())
        return carry2

      lax.fori_loop(0, n_sg, sgroup, 0)
      plsc.subcore_barrier()
      pltpu.sync_copy(acc.at[pl.ds(t * STEPB, SZB)],
                      out_hbm.at[pl.ds(base + t * STEPB, SZB)])
      plsc.subcore_barrier()
      return carry

    lax.fori_loop(0, ppc, one_pass, 0)

  return functools.partial(
      pl.kernel,
      out_type=jax.ShapeDtypeStruct((n, width), _f32),
      mesh=_MESH,
      scratch_types=scratch,
  )(body)


_BR = 2000


def _tc(body, n, dout, in_arrays, in_shapes):
  specs = []
  for s in in_shapes:
    if s[0] is None:
      specs.append(pl.BlockSpec(s[1], lambda i: (0, 0)))
    else:
      specs.append(pl.BlockSpec((_BR, s[1]), lambda i: (i, 0)))
  return pl.pallas_call(
      body,
      grid=(n // _BR,),
      in_specs=specs,
      out_specs=pl.BlockSpec((_BR, dout), lambda i: (i, 0)),
      out_shape=jax.ShapeDtypeStruct((n, dout), _f32),
  )(*in_arrays)


def _mm(x, w, dinv, pad_to=None):
  n, kdim = x.shape
  dout = w.shape[1]
  width = pad_to or dout

  def body(x_ref, w_ref, d_ref, o_ref):
    y = jnp.dot(x_ref[...], w_ref[...],
                preferred_element_type=_f32) * d_ref[...]
    if width > dout:
      y = jnp.concatenate([y, jnp.zeros((y.shape[0], width - dout), _f32)],
                          axis=1)
    o_ref[...] = y

  return _tc(body, n, width, (x, w, dinv),
             ((0, kdim), (None, (kdim, dout)), (0, 1)))


def _post(a, dinv, b):
  n, d = a.shape

  def body(a_ref, d_ref, b_ref, o_ref):
    o_ref[...] = jnp.maximum(a_ref[...] * d_ref[...] + b_ref[...], 0.0)

  return _tc(body, n, d, (a, dinv, b), ((0, d), (0, 1), (None, (1, d))))


def _post_mm(a, dinv, b, w2, real_d=None, pad_to=None):
  n, d = a.shape
  rd = real_d or d
  kd = w2.shape[0]
  dout = w2.shape[1]
  width = pad_to or dout

  def body(a_ref, d_ref, b_ref, w_ref, o_ref):
    h = jnp.maximum(a_ref[...][:, :rd] * d_ref[...] + b_ref[...], 0.0)
    y = jnp.dot(h, w_ref[...], preferred_element_type=_f32) * d_ref[...]
    if width > dout:
      y = jnp.concatenate([y, jnp.zeros((y.shape[0], width - dout), _f32)],
                          axis=1)
    o_ref[...] = y

  return _tc(body, n, width, (a, dinv, b, w2),
             ((0, d), (0, 1), (None, (1, rd)), (None, (kd, dout))))


def _out2(a_first, b_first, a_second, b_second, dinv, real_d1=None):
  n, d1 = a_first.shape
  rd1 = real_d1 or d1
  d2 = a_second.shape[1]

  def body(a1_ref, b1_ref, a2_ref, b2_ref, d_ref, o_ref):
    y1 = jnp.maximum(a1_ref[...][:, :rd1] * d_ref[...] + b1_ref[...], 0.0)
    y2 = jnp.maximum(a2_ref[...] * d_ref[...] + b2_ref[...], 0.0)
    o_ref[...] = jnp.concatenate([y1, y2], axis=1)

  return _tc(body, n, rd1 + d2, (a_first, b_first, a_second, b_second, dinv),
             ((0, d1), (None, (1, rd1)), (0, d2), (None, (1, d2)), (0, 1)))


def _dinv1(deg):
  n = deg.shape[0]

  def body(deg_ref, o_ref):
    o_ref[...] = lax.rsqrt(jnp.maximum(deg_ref[...][:, :1], 1e-12))

  return _tc(body, n, 1, (deg,), ((0, 128),))


def kernel(feature_v, edge_index, feature_e, trans_edge_index,
           W1v, b1v, W1e, b1e, Ws1, bs1, Ws2, bs2,
           W2v, b2v, W3v, b3v, W2e, b2e, W3e, b3e):
  ei = edge_index.astype(_i32)
  te = trans_edge_index.astype(_i32)
  src_v, dst_v = ei[0], ei[1]
  src_e, dst_e = te[0], te[1]

  rv = 5000
  pv = NV // rv
  bucket_v = dst_v // rv
  order_v = jnp.argsort(bucket_v)
  sv = src_v[order_v]
  dstv_s = dst_v[order_v]
  dvl = dstv_s - (dstv_s // rv) * rv
  bnd_v = jnp.searchsorted(dstv_s // rv,
                           jnp.arange(pv + 1, dtype=_i32)).astype(_i32)
  bnd_v96 = jnp.concatenate([bnd_v, jnp.full((96 - (pv + 1),), EV, _i32)])
  erv = EV // CH + 64
  padv = erv * CH - EV
  sv_p = jnp.concatenate([sv, jnp.zeros((padv,), _i32)])
  dvl_p = jnp.concatenate([dvl, jnp.full((padv,), rv, _i32)])

  r = 5000
  p = NE // r
  bucket = dst_e // r
  order = jnp.argsort(bucket)
  se = src_e[order]
  dst_s = dst_e[order]
  de = dst_s - (dst_s // r) * r
  bnd = jnp.searchsorted(dst_s // r,
                         jnp.arange(p + 1, dtype=_i32)).astype(_i32)
  bnd64 = jnp.concatenate([bnd, jnp.full((96 - (p + 1),), ET, _i32)])
  er = ET // CH + 64
  padl = er * CH - ET
  se_p = jnp.concatenate([se, jnp.zeros((padl,), _i32)])
  de_p = jnp.concatenate([de, jnp.full((padl,), r, _i32)])
  ones_sc = jnp.ones((CH, 128), _f32)

  b1v_ = b1v.reshape(1, -1)
  b1e_ = b1e.reshape(1, -1)
  bs1_ = bs1.reshape(1, -1)
  bs2_ = bs2.reshape(1, -1)
  b2v_ = b2v.reshape(1, -1)
  b3v_ = b3v.reshape(1, -1)
  b2e_ = b2e.reshape(1, -1)
  b3e_ = b3e.reshape(1, -1)

  agg_v = _sc_agg_binned(NV, 128, rv, pv, True)
  deg_v_k = _sc_agg_binned(NV, 0, rv, pv, False)
  agg_e128 = _sc_agg_binned(NE, 128, r, p, True)
  deg_e_k = _sc_agg_binned(NE, 0, r, p, False)

  degv = deg_v_k(sv_p, dvl_p, bnd_v96, ones_sc)
  dege = deg_e_k(se_p, de_p, bnd64, ones_sc)
  dinv_v = _dinv1(degv)
  dinv_e = _dinv1(dege)

  g1 = _mm(feature_v, W1v, dinv_v)
  A1 = agg_v(g1, sv_p, dvl_p, bnd_v96)
  fv = _post(A1, dinv_v, b1v_)

  gs1 = _mm(fv, Ws1, dinv_v)
  As1 = agg_v(gs1, sv_p, dvl_p, bnd_v96)
  gs2 = _post_mm(As1, dinv_v, bs1_, Ws2)
  As2 = agg_v(gs2, sv_p, dvl_p, bnd_v96)

  g2 = _mm(fv, W2v, dinv_v)
  A2 = agg_v(g2, sv_p, dvl_p, bnd_v96)
  g3 = _post_mm(A2, dinv_v, b2v_, W3v)
  A3 = agg_v(g3, sv_p, dvl_p, bnd_v96)

  fv_out = _out2(A3, b3v_, As2, bs2_, dinv_v)

  ge1 = _mm(feature_e, W1e, dinv_e)
  Ae1 = agg_e128(ge1, se_p, de_p, bnd64)
  fe = _post(Ae1, dinv_e, b1e_)

  ges1 = _mm(fe, Ws1, dinv_e)
  Aes1 = agg_e128(ges1, se_p, de_p, bnd64)
  ges2 = _post_mm(Aes1, dinv_e, bs1_, Ws2)
  Aes2 = agg_e128(ges2, se_p, de_p, bnd64)

  ge2 = _mm(fe, W2e, dinv_e, pad_to=128)
  Ae2 = agg_e128(ge2, se_p, de_p, bnd64)
  ge3 = _post_mm(Ae2, dinv_e, b2e_, W3e, real_d=64, pad_to=128)
  Ae3 = agg_e128(ge3, se_p, de_p, bnd64)

  fe_out = _out2(Ae3, b3e_, Aes2, bs2_, dinv_e, real_d1=64)

  return fv_out, fe_out
